# baseline (speedup 1.0000x reference)
import jax
import jax.numpy as jnp
from jax import lax
from jax.experimental import pallas as pl
from jax.experimental.pallas import tpu as pltpu
from jax.experimental.pallas import tpu_sc as plsc

MAX_T = 8192
DM = 1024
NB = 4

NC = 2
NS = 16
NW = NC * NS

TT = 16
GROUP = 1
N_GROUPS = NB // GROUP
T_PER_W = MAX_T // NW
N_TSTEPS = T_PER_W // TT
N_STEPS = N_TSTEPS * N_GROUPS
LANES = 16
VEC_PER_ROW = DM // LANES
N_VEC = TT * VEC_PER_ROW
UNROLL = 8

R_X = 5
R_E = 2
PF = 2
KEEP_OUT = 2


def _sc_add(x_hbm, emb_hbm, out_hbm, emb_v, x_v, *sems):
    wid = lax.axis_index("s") * NC + lax.axis_index("c")
    base = wid * T_PER_W

    sem_e = sems[:R_E]
    sem_i = sems[R_E:R_E + R_X]
    sem_o = sems[R_E + R_X:]

    def start_in(s):
        k, g = s // N_GROUPS, s % N_GROUPS
        p, es = s % R_X, k % R_E
        ts = base + k * TT
        he = None
        if g == 0:
            he = pltpu.async_copy(emb_hbm.at[pl.ds(ts, TT)], emb_v.at[es],
                                  sem_e[es])
        hx = pltpu.async_copy(x_hbm.at[pl.ds(GROUP * g, GROUP),
                                       pl.ds(ts, TT)],
                              x_v.at[p], sem_i[p])
        return he, hx

    pend_in = [start_in(s0) for s0 in range(PF)]
    pend_out = []

    for s in range(N_STEPS):
        k, g = s // N_GROUPS, s % N_GROUPS
        p, es = s % R_X, k % R_E
        while len(pend_out) > KEEP_OUT:
            pend_out.pop(0).wait()
        if s + PF < N_STEPS:
            pend_in.append(start_in(s + PF))
        he, hx = pend_in.pop(0)
        if he is not None:
            he.wait()
        hx.wait()

        @plsc.parallel_loop(0, N_VEC, unroll=UNROLL)
        def _(j):
            r = j // VEC_PER_ROW
            sl = pl.ds((j % VEC_PER_ROW) * LANES, LANES)
            e = emb_v[es, r, sl]
            for bb in range(GROUP):
                plsc.addupdate(x_v.at[p, bb, r, sl], e)

        ts = base + k * TT
        pend_out.append(
            pltpu.async_copy(x_v.at[p],
                             out_hbm.at[pl.ds(GROUP * g, GROUP),
                                        pl.ds(ts, TT)],
                             sem_o[p]))

    for h in pend_out:
        h.wait()


@jax.jit
def _sc_kernel(x, emb):
    mesh = plsc.VectorSubcoreMesh(core_axis_name="c", subcore_axis_name="s")
    return pl.kernel(
        _sc_add,
        mesh=mesh,
        out_type=jax.ShapeDtypeStruct((NB, MAX_T, DM), jnp.float32),
        scratch_types=[
            pltpu.VMEM((R_E, TT, DM), jnp.float32),
            pltpu.VMEM((R_X, GROUP, TT, DM), jnp.float32),
        ] + [pltpu.SemaphoreType.DMA] * (R_E + 2 * R_X),
        compiler_params=pltpu.CompilerParams(use_tc_tiling_on_sc=True),
    )(x, emb)


def kernel(x, emb):
    return _sc_kernel(x, emb)


# speedup vs baseline: 1.0058x; 1.0058x over previous
import jax
import jax.numpy as jnp
from jax import lax
from jax.experimental import pallas as pl
from jax.experimental.pallas import tpu as pltpu
from jax.experimental.pallas import tpu_sc as plsc

MAX_T = 8192
DM = 1024
NB = 4

NC = 2
NS = 16
NW = NC * NS

TT = 8
GROUP = 2
N_GROUPS = NB // GROUP
T_PER_W = MAX_T // NW
N_TSTEPS = T_PER_W // TT
N_STEPS = N_TSTEPS * N_GROUPS
LANES = 16
VEC_PER_ROW = DM // LANES
N_VEC = TT * VEC_PER_ROW
UNROLL = 8

R_X = 6
R_E = 3
PF = 3
KEEP_OUT = 2


def _sc_add(x_hbm, emb_hbm, out_hbm, emb_v, x_v,
            sem_e0, sem_e1, sem_e2, sem_i0, sem_i1, sem_i2,
            sem_i3, sem_i4, sem_i5, sem_o0, sem_o1, sem_o2,
            sem_o3, sem_o4, sem_o5):
    wid = lax.axis_index("s") * NC + lax.axis_index("c")
    base = wid * T_PER_W

    sem_e = (sem_e0, sem_e1, sem_e2)
    sem_i = (sem_i0, sem_i1, sem_i2, sem_i3, sem_i4, sem_i5)
    sem_o = (sem_o0, sem_o1, sem_o2, sem_o3, sem_o4, sem_o5)

    def start_in(s):
        k, g = s // N_GROUPS, s % N_GROUPS
        p, es = s % R_X, k % R_E
        ts = base + k * TT
        he = None
        if g == 0:
            he = pltpu.async_copy(emb_hbm.at[pl.ds(ts, TT)], emb_v.at[es],
                                  sem_e[es])
        hx = pltpu.async_copy(x_hbm.at[pl.ds(GROUP * g, GROUP),
                                       pl.ds(ts, TT)],
                              x_v.at[p], sem_i[p])
        return he, hx

    pend_in = [start_in(s0) for s0 in range(PF)]
    pend_out = []

    for s in range(N_STEPS):
        k, g = s // N_GROUPS, s % N_GROUPS
        p, es = s % R_X, k % R_E
        while len(pend_out) > KEEP_OUT:
            pend_out.pop(0).wait()
        if s + PF < N_STEPS:
            pend_in.append(start_in(s + PF))
        he, hx = pend_in.pop(0)
        if he is not None:
            he.wait()
        hx.wait()

        @plsc.parallel_loop(0, N_VEC, unroll=UNROLL)
        def _(j):
            r = j // VEC_PER_ROW
            sl = pl.ds((j % VEC_PER_ROW) * LANES, LANES)
            e = emb_v[es, r, sl]
            for bb in range(GROUP):
                plsc.addupdate(x_v.at[p, bb, r, sl], e)

        ts = base + k * TT
        pend_out.append(
            pltpu.async_copy(x_v.at[p],
                             out_hbm.at[pl.ds(GROUP * g, GROUP),
                                        pl.ds(ts, TT)],
                             sem_o[p]))

    for h in pend_out:
        h.wait()


@jax.jit
def _sc_kernel(x, emb):
    mesh = plsc.VectorSubcoreMesh(core_axis_name="c", subcore_axis_name="s")
    return pl.kernel(
        _sc_add,
        mesh=mesh,
        out_type=jax.ShapeDtypeStruct((NB, MAX_T, DM), jnp.float32),
        scratch_types=[
            pltpu.VMEM((R_E, TT, DM), jnp.float32),
            pltpu.VMEM((R_X, GROUP, TT, DM), jnp.float32),
        ] + [pltpu.SemaphoreType.DMA] * (R_E + 2 * R_X),
        compiler_params=pltpu.CompilerParams(use_tc_tiling_on_sc=True),
    )(x, emb)


def kernel(x, emb):
    return _sc_kernel(x, emb)
